# baseline (speedup 1.0000x reference)
import functools

import jax
import jax.numpy as jnp
from jax import lax
from jax.experimental import pallas as pl
from jax.experimental.pallas import tpu as pltpu
from jax.experimental.pallas import tpu_sc as plsc

N = 10000
E = 320000
F = 128
H = 256
G = 64
L = 16

NC = 2
NS = 16
NW = NC * NS
CHUNK = 128
NCHUNKS = E // CHUNK
CPT = 80
CPT_LAST = NCHUNKS - (NW - 1) * CPT
NP = 10240
ROWS_PER_SUB = NP // NS

_mesh = plsc.VectorSubcoreMesh(core_axis_name="c", subcore_axis_name="s")
_sc_params = pltpu.CompilerParams(use_tc_tiling_on_sc=False)

_f32 = jnp.float32


def _copy_my_chunks(src2_hbm, dst_v, wid):
    r0 = wid * CPT

    @pl.when(wid < NW - 1)
    def _():
        pltpu.sync_copy(src2_hbm.at[pl.ds(r0, CPT)], dst_v.at[pl.ds(0, CPT)])

    @pl.when(wid == NW - 1)
    def _():
        pltpu.sync_copy(src2_hbm.at[pl.ds((NW - 1) * CPT, CPT_LAST)],
                        dst_v.at[pl.ds(0, CPT_LAST)])


def _my_num_chunks(wid):
    return jnp.where(wid < NW - 1, CPT, CPT_LAST)


@functools.partial(
    pl.kernel,
    out_type=jax.ShapeDtypeStruct((NC, N, L), _f32),
    mesh=_mesh,
    scratch_types=[
        pltpu.VMEM((CPT, CHUNK), jnp.int32),
        pltpu.VMEM((CHUNK, L), _f32),
        pltpu.VMEM_SHARED((N, L), _f32),
        pltpu.SemaphoreType.DMA,
    ],
    compiler_params=_sc_params,
)
def _deg_kernel(dst2_hbm, zeros_hbm, ones_hbm, out, dst_idx, ones_v, acc, sem):
    c = lax.axis_index("c")
    s = lax.axis_index("s")
    wid = c * NS + s

    @pl.when(s == 0)
    def _():
        pltpu.sync_copy(zeros_hbm, acc)

    pltpu.sync_copy(ones_hbm, ones_v)
    _copy_my_chunks(dst2_hbm, dst_idx, wid)
    plsc.subcore_barrier()

    nr = _my_num_chunks(wid)

    def fire(i, carry):
        pltpu.async_copy(ones_v, acc.at[dst_idx.at[i]], sem, add=True)
        return carry

    lax.fori_loop(0, nr, fire, 0)

    def drain(i, carry):
        pltpu.make_async_copy(ones_v, acc.at[dst_idx.at[0]], sem).wait()
        return carry

    lax.fori_loop(0, nr, drain, 0)
    plsc.subcore_barrier()

    @pl.when(s == 0)
    def _():
        pltpu.sync_copy(acc, out.at[c])


FH = F // 2
CPS = NCHUNKS // NS
CPS_LAST = NCHUNKS - (NS - 1) * CPS


@functools.partial(
    pl.kernel,
    out_type=jax.ShapeDtypeStruct((NC, NP, FH), _f32),
    mesh=_mesh,
    scratch_types=[
        pltpu.VMEM((CPS_LAST, CHUNK), jnp.int32),
        pltpu.VMEM((CPS_LAST, CHUNK), jnp.int32),
        pltpu.VMEM((4, CHUNK, FH), _f32),
        pltpu.VMEM_SHARED((NP, FH), _f32),
        pltpu.SemaphoreType.DMA,
        pltpu.SemaphoreType.DMA,
    ],
    compiler_params=_sc_params,
)
def _agg_kernel(y2_hbm, srcb_hbm, dst2_hbm, zeros_hbm, out,
                src_idx, dst_idx, bufs, acc, gsem, ssem):
    c = lax.axis_index("c")
    s = lax.axis_index("s")
    row0 = s * ROWS_PER_SUB

    pltpu.sync_copy(zeros_hbm.at[pl.ds(row0, ROWS_PER_SUB)],
                    acc.at[pl.ds(row0, ROWS_PER_SUB)])
    plsc.subcore_barrier()

    def gather_pair(p, b0):
        i0 = 2 * p
        pltpu.async_copy(y2_hbm.at[src_idx.at[i0]], bufs.at[b0], gsem)
        pltpu.async_copy(y2_hbm.at[src_idx.at[i0 + 1]], bufs.at[b0 + 1], gsem)

    def drain_g(n):
        def one(j, carry):
            pltpu.make_async_copy(y2_hbm.at[src_idx.at[0]], bufs.at[0],
                                  gsem).wait()
            return carry
        lax.fori_loop(0, n, one, 0)

    def drain_s(n):
        def one(j, carry):
            pltpu.make_async_copy(bufs.at[0], acc.at[dst_idx.at[0]],
                                  ssem).wait()
            return carry
        lax.fori_loop(0, n, one, 0)

    def scatter_pair(p, b0):
        i0 = 2 * p
        pltpu.async_copy(bufs.at[b0], acc.at[dst_idx.at[i0]], ssem, add=True)
        pltpu.async_copy(bufs.at[b0 + 1], acc.at[dst_idx.at[i0 + 1]], ssem,
                         add=True)

    def run(nchunks):
        r0 = s * CPS
        pltpu.sync_copy(srcb_hbm.at[c, pl.ds(r0, nchunks)],
                        src_idx.at[pl.ds(0, nchunks)])
        pltpu.sync_copy(dst2_hbm.at[pl.ds(r0, nchunks)],
                        dst_idx.at[pl.ds(0, nchunks)])
        npairs = nchunks // 2
        gather_pair(0, 0)

        def it(g, carry):
            b0 = (g % 2) * 2
            drain_g(2)

            @pl.when(g > 0)
            def _():
                drain_s(2)

            @pl.when(g + 1 < npairs)
            def _():
                gather_pair(g + 1, 2 - b0)

            scatter_pair(g, b0)
            return carry

        lax.fori_loop(0, npairs, it, 0)
        drain_s(2)

    @pl.when(s < NS - 1)
    def _():
        run(CPS)

    @pl.when(s == NS - 1)
    def _():
        run(CPS_LAST)

    plsc.subcore_barrier()
    pltpu.sync_copy(acc.at[pl.ds(row0, ROWS_PER_SUB)],
                    out.at[c, pl.ds(row0, ROWS_PER_SUB)])


@functools.partial(
    pl.kernel,
    out_type=jax.ShapeDtypeStruct((NC, N, L), _f32),
    mesh=_mesh,
    scratch_types=[
        pltpu.VMEM((CPT, CHUNK), jnp.int32),
        pltpu.VMEM((CPT, CHUNK), jnp.int32),
        pltpu.VMEM((4, CHUNK, L), _f32),
        pltpu.VMEM_SHARED((N, L), _f32),
        pltpu.SemaphoreType.DMA,
        pltpu.SemaphoreType.DMA,
    ],
    compiler_params=_sc_params,
)
def _l2_kernel(zp_hbm, src2_hbm, dst2_hbm, zeros_hbm, out,
               src_idx, dst_idx, bufs, acc, sem0, sem1):
    c = lax.axis_index("c")
    s = lax.axis_index("s")
    wid = c * NS + s

    @pl.when(s == 0)
    def _():
        pltpu.sync_copy(zeros_hbm, acc)

    _copy_my_chunks(src2_hbm, src_idx, wid)
    _copy_my_chunks(dst2_hbm, dst_idx, wid)
    plsc.subcore_barrier()

    nr = _my_num_chunks(wid)

    def gather_pair(p, b0):
        i0 = 2 * p
        pltpu.async_copy(zp_hbm.at[src_idx.at[i0]], bufs.at[b0], sem0)
        pltpu.async_copy(zp_hbm.at[src_idx.at[i0 + 1]], bufs.at[b0 + 1], sem0)

    def scatter_pair(p, b0):
        i0 = 2 * p
        pltpu.async_copy(bufs.at[b0], acc.at[dst_idx.at[i0]], sem1, add=True)
        pltpu.async_copy(bufs.at[b0 + 1], acc.at[dst_idx.at[i0 + 1]], sem1,
                         add=True)

    def drain(make, n):
        def one(j, carry):
            make().wait()
            return carry
        lax.fori_loop(0, n, one, 0)

    def drain_g(n):
        drain(lambda: pltpu.make_async_copy(zp_hbm.at[src_idx.at[0]],
                                            bufs.at[0], sem0), n)

    def drain_s(n):
        drain(lambda: pltpu.make_async_copy(bufs.at[0],
                                            acc.at[dst_idx.at[0]], sem1), n)

    npairs = nr // 2
    gather_pair(0, 0)

    def it(g, carry):
        b0 = (g % 2) * 2
        drain_g(2)

        @pl.when(g > 0)
        def _():
            drain_s(2)

        @pl.when(g + 1 < npairs)
        def _():
            gather_pair(g + 1, 2 - b0)

        scatter_pair(g, b0)
        return carry

    lax.fori_loop(0, npairs, it, 0)
    drain_s(2)
    plsc.subcore_barrier()

    @pl.when(s == 0)
    def _():
        pltpu.sync_copy(acc, out.at[c])


BLK = 1000


def _prep_body(deg_ref, x_ref, dinv_ref, y2_ref):
    deg = deg_ref[0, :, 0:1] + deg_ref[1, :, 0:1] + 2.0
    dinv = lax.rsqrt(deg)
    dinv_ref[...] = dinv
    xb = x_ref[...] * dinv
    y2_ref[0] = xb[:, :FH]
    y2_ref[1] = xb[:, FH:]


_prep = pl.pallas_call(
    _prep_body,
    grid=(N // BLK,),
    in_specs=[
        pl.BlockSpec((NC, BLK, L), lambda i: (0, i, 0)),
        pl.BlockSpec((BLK, F), lambda i: (i, 0)),
    ],
    out_specs=[
        pl.BlockSpec((BLK, 1), lambda i: (i, 0)),
        pl.BlockSpec((NC, BLK, FH), lambda i: (0, i, 0)),
    ],
    out_shape=[
        jax.ShapeDtypeStruct((N, 1), _f32),
        jax.ShapeDtypeStruct((NC, N, FH), _f32),
    ],
)


def _dense_body(dinv_ref, x_ref, a_ref, W1_ref, b1_ref, W2_ref,
                z_ref, zp_ref):
    dinv = dinv_ref[...]
    a = jnp.concatenate([a_ref[0], a_ref[1]], axis=1)
    t = dinv * a + (2.0 * dinv * dinv) * x_ref[...]
    h = jnp.maximum(
        jnp.dot(t, W1_ref[...], preferred_element_type=_f32) + b1_ref[...],
        0.0)
    z = jnp.dot(h, W2_ref[...], preferred_element_type=_f32)
    z_ref[...] = z
    zp_ref[...] = jnp.broadcast_to(z * dinv, (z.shape[0], L))


_dense = pl.pallas_call(
    _dense_body,
    grid=(N // BLK,),
    in_specs=[
        pl.BlockSpec((BLK, 1), lambda i: (i, 0)),
        pl.BlockSpec((BLK, F), lambda i: (i, 0)),
        pl.BlockSpec((NC, BLK, FH), lambda i: (0, i, 0)),
        pl.BlockSpec((F, H), lambda i: (0, 0)),
        pl.BlockSpec((1, H), lambda i: (0, 0)),
        pl.BlockSpec((H, 1), lambda i: (0, 0)),
    ],
    out_specs=[
        pl.BlockSpec((BLK, 1), lambda i: (i, 0)),
        pl.BlockSpec((BLK, L), lambda i: (i, 0)),
    ],
    out_shape=[
        jax.ShapeDtypeStruct((N, 1), _f32),
        jax.ShapeDtypeStruct((N, L), _f32),
    ],
)


def _final_body(q_ref, dinv_ref, z_ref, b2_ref, batch_ref, pooled_ref):
    dinv = dinv_ref[...]
    q = q_ref[0, :, 0:1] + q_ref[1, :, 0:1]
    out = dinv * q + (2.0 * dinv * dinv) * z_ref[...] + b2_ref[0, 0]
    gids = lax.broadcasted_iota(jnp.int32, (1, G), 1)
    onehot = (batch_ref[...] == gids).astype(_f32)
    cat = jnp.concatenate([out, jnp.ones_like(out)], axis=1)
    r = lax.dot_general(onehot, cat, (((0,), (0,)), ((), ())),
                        preferred_element_type=_f32)
    pooled_ref[...] = r[:, 0:1] / jnp.maximum(r[:, 1:2], 1.0)


_final = pl.pallas_call(
    _final_body,
    out_shape=jax.ShapeDtypeStruct((G, 1), _f32),
)


def kernel(x, edge_index, edge_attr, batch, W1, b1, W2, b2):
    del edge_attr
    src2 = edge_index[0].reshape(NCHUNKS, CHUNK)
    dst2 = edge_index[1].reshape(NCHUNKS, CHUNK)
    srcb = jnp.stack([src2, src2 + N])
    zeros_nf = jnp.zeros((NP, FH), _f32)
    zeros_nl = jnp.zeros((N, L), _f32)
    ones_cl = jnp.ones((CHUNK, L), _f32)

    deg = _deg_kernel(dst2, zeros_nl, ones_cl)
    dinv, y2 = _prep(deg, x)
    acc = _agg_kernel(y2.reshape(NC * N, FH), srcb, dst2, zeros_nf)
    z, zp = _dense(dinv, x, acc, W1, b1.reshape(1, H), W2)
    q = _l2_kernel(zp, src2, dst2, zeros_nl)
    pooled = _final(q, dinv, z, b2.reshape(1, 1), batch.reshape(N, 1))
    return pooled


# speedup vs baseline: 1.0204x; 1.0204x over previous
import functools

import jax
import jax.numpy as jnp
from jax import lax
from jax.experimental import pallas as pl
from jax.experimental.pallas import tpu as pltpu
from jax.experimental.pallas import tpu_sc as plsc

N = 10000
E = 320000
F = 128
H = 256
G = 64
L = 16

NC = 2
NS = 16
NW = NC * NS
CHUNK = 128
NCHUNKS = E // CHUNK
CPT = 80
CPT_LAST = NCHUNKS - (NW - 1) * CPT
NP = 10240
ROWS_PER_SUB = NP // NS

_mesh = plsc.VectorSubcoreMesh(core_axis_name="c", subcore_axis_name="s")
_sc_params = pltpu.CompilerParams(use_tc_tiling_on_sc=False)

_f32 = jnp.float32


def _copy_my_chunks(src2_hbm, dst_v, wid):
    r0 = wid * CPT

    @pl.when(wid < NW - 1)
    def _():
        pltpu.sync_copy(src2_hbm.at[pl.ds(r0, CPT)], dst_v.at[pl.ds(0, CPT)])

    @pl.when(wid == NW - 1)
    def _():
        pltpu.sync_copy(src2_hbm.at[pl.ds((NW - 1) * CPT, CPT_LAST)],
                        dst_v.at[pl.ds(0, CPT_LAST)])


def _my_num_chunks(wid):
    return jnp.where(wid < NW - 1, CPT, CPT_LAST)


@functools.partial(
    pl.kernel,
    out_type=jax.ShapeDtypeStruct((NC, N, L), _f32),
    mesh=_mesh,
    scratch_types=[
        pltpu.VMEM((CPT, CHUNK), jnp.int32),
        pltpu.VMEM((CHUNK, L), _f32),
        pltpu.VMEM_SHARED((N, L), _f32),
        pltpu.SemaphoreType.DMA,
    ],
    compiler_params=_sc_params,
)
def _deg_kernel(dst2_hbm, zeros_hbm, ones_hbm, out, dst_idx, ones_v, acc, sem):
    c = lax.axis_index("c")
    s = lax.axis_index("s")
    wid = c * NS + s

    @pl.when(s == 0)
    def _():
        pltpu.sync_copy(zeros_hbm, acc)

    pltpu.sync_copy(ones_hbm, ones_v)
    _copy_my_chunks(dst2_hbm, dst_idx, wid)
    plsc.subcore_barrier()

    nr = _my_num_chunks(wid)

    def fire(i, carry):
        pltpu.async_copy(ones_v, acc.at[dst_idx.at[i]], sem, add=True)
        return carry

    lax.fori_loop(0, nr, fire, 0)

    def drain(i, carry):
        pltpu.make_async_copy(ones_v, acc.at[dst_idx.at[0]], sem).wait()
        return carry

    lax.fori_loop(0, nr, drain, 0)
    plsc.subcore_barrier()

    @pl.when(s == 0)
    def _():
        pltpu.sync_copy(acc, out.at[c])


FH = F // 2
CPS = NCHUNKS // NS
CPS_LAST = NCHUNKS - (NS - 1) * CPS


@functools.partial(
    pl.kernel,
    out_type=jax.ShapeDtypeStruct((NC, NP, FH), _f32),
    mesh=_mesh,
    scratch_types=[
        pltpu.VMEM((CPS_LAST, CHUNK), jnp.int32),
        pltpu.VMEM((CPS_LAST, CHUNK), jnp.int32),
        pltpu.VMEM((4, CHUNK, FH), _f32),
        pltpu.VMEM_SHARED((NP, FH), _f32),
        pltpu.SemaphoreType.DMA,
        pltpu.SemaphoreType.DMA,
    ],
    compiler_params=_sc_params,
)
def _agg_kernel(y2_hbm, srcb_hbm, dst2_hbm, zeros_hbm, out,
                src_idx, dst_idx, bufs, acc, gsem, ssem):
    c = lax.axis_index("c")
    s = lax.axis_index("s")
    row0 = s * ROWS_PER_SUB

    pltpu.sync_copy(zeros_hbm.at[pl.ds(row0, ROWS_PER_SUB)],
                    acc.at[pl.ds(row0, ROWS_PER_SUB)])
    plsc.subcore_barrier()

    def gather_pair(p, b0):
        i0 = 2 * p
        pltpu.async_copy(y2_hbm.at[src_idx.at[i0]], bufs.at[b0], gsem)
        pltpu.async_copy(y2_hbm.at[src_idx.at[i0 + 1]], bufs.at[b0 + 1], gsem)

    def drain_g(n):
        def one(j, carry):
            pltpu.make_async_copy(y2_hbm.at[src_idx.at[0]], bufs.at[0],
                                  gsem).wait()
            return carry
        lax.fori_loop(0, n, one, 0)

    def drain_s(n):
        def one(j, carry):
            pltpu.make_async_copy(bufs.at[0], acc.at[dst_idx.at[0]],
                                  ssem).wait()
            return carry
        lax.fori_loop(0, n, one, 0)

    def scatter_pair(p, b0):
        i0 = 2 * p
        pltpu.async_copy(bufs.at[b0], acc.at[dst_idx.at[i0]], ssem, add=True)
        pltpu.async_copy(bufs.at[b0 + 1], acc.at[dst_idx.at[i0 + 1]], ssem,
                         add=True)

    def run(nchunks):
        r0 = s * CPS
        pltpu.sync_copy(srcb_hbm.at[c, pl.ds(r0, nchunks)],
                        src_idx.at[pl.ds(0, nchunks)])
        pltpu.sync_copy(dst2_hbm.at[pl.ds(r0, nchunks)],
                        dst_idx.at[pl.ds(0, nchunks)])
        npairs = nchunks // 2
        gather_pair(0, 0)

        def it(g, carry):
            b0 = (g % 2) * 2
            drain_g(2)

            @pl.when(g > 0)
            def _():
                drain_s(2)

            @pl.when(g + 1 < npairs)
            def _():
                gather_pair(g + 1, 2 - b0)

            scatter_pair(g, b0)
            return carry

        lax.fori_loop(0, npairs, it, 0)
        drain_s(2)

    @pl.when(s < NS - 1)
    def _():
        run(CPS)

    @pl.when(s == NS - 1)
    def _():
        run(CPS_LAST)

    plsc.subcore_barrier()
    pltpu.sync_copy(acc.at[pl.ds(row0, ROWS_PER_SUB)],
                    out.at[c, pl.ds(row0, ROWS_PER_SUB)])


@functools.partial(
    pl.kernel,
    out_type=jax.ShapeDtypeStruct((NC, N, L), _f32),
    mesh=_mesh,
    scratch_types=[
        pltpu.VMEM((CPT, CHUNK), jnp.int32),
        pltpu.VMEM((CPT, CHUNK), jnp.int32),
        pltpu.VMEM((4, CHUNK, L), _f32),
        pltpu.VMEM_SHARED((N, L), _f32),
        pltpu.SemaphoreType.DMA,
        pltpu.SemaphoreType.DMA,
    ],
    compiler_params=_sc_params,
)
def _l2_kernel(zp_hbm, src2_hbm, dst2_hbm, zeros_hbm, out,
               src_idx, dst_idx, bufs, acc, sem0, sem1):
    c = lax.axis_index("c")
    s = lax.axis_index("s")
    wid = c * NS + s

    @pl.when(s == 0)
    def _():
        pltpu.sync_copy(zeros_hbm, acc)

    _copy_my_chunks(src2_hbm, src_idx, wid)
    _copy_my_chunks(dst2_hbm, dst_idx, wid)
    plsc.subcore_barrier()

    nr = _my_num_chunks(wid)

    def group(g, carry):
        i0 = 4 * g
        for j in range(4):
            pltpu.async_copy(zp_hbm.at[src_idx.at[i0 + j]], bufs.at[j], sem0)
        for j in range(4):
            pltpu.make_async_copy(zp_hbm.at[src_idx.at[i0 + j]], bufs.at[j],
                                  sem0).wait()
        for j in range(4):
            pltpu.async_copy(bufs.at[j], acc.at[dst_idx.at[i0 + j]], sem1,
                             add=True)
        for j in range(4):
            pltpu.make_async_copy(bufs.at[j], acc.at[dst_idx.at[i0 + j]],
                                  sem1).wait()
        return carry

    lax.fori_loop(0, nr // 4, group, 0)
    plsc.subcore_barrier()

    @pl.when(s == 0)
    def _():
        pltpu.sync_copy(acc, out.at[c])


BLK = 1000


def _prep_body(deg_ref, x_ref, dinv_ref, y2_ref):
    deg = deg_ref[0, :, 0:1] + deg_ref[1, :, 0:1] + 2.0
    dinv = lax.rsqrt(deg)
    dinv_ref[...] = dinv
    xb = x_ref[...] * dinv
    y2_ref[0] = xb[:, :FH]
    y2_ref[1] = xb[:, FH:]


_prep = pl.pallas_call(
    _prep_body,
    grid=(N // BLK,),
    in_specs=[
        pl.BlockSpec((NC, BLK, L), lambda i: (0, i, 0)),
        pl.BlockSpec((BLK, F), lambda i: (i, 0)),
    ],
    out_specs=[
        pl.BlockSpec((BLK, 1), lambda i: (i, 0)),
        pl.BlockSpec((NC, BLK, FH), lambda i: (0, i, 0)),
    ],
    out_shape=[
        jax.ShapeDtypeStruct((N, 1), _f32),
        jax.ShapeDtypeStruct((NC, N, FH), _f32),
    ],
)


def _dense_body(dinv_ref, x_ref, a_ref, W1_ref, b1_ref, W2_ref,
                z_ref, zp_ref):
    dinv = dinv_ref[...]
    a = jnp.concatenate([a_ref[0], a_ref[1]], axis=1)
    t = dinv * a + (2.0 * dinv * dinv) * x_ref[...]
    h = jnp.maximum(
        jnp.dot(t, W1_ref[...], preferred_element_type=_f32) + b1_ref[...],
        0.0)
    z = jnp.dot(h, W2_ref[...], preferred_element_type=_f32)
    z_ref[...] = z
    zp_ref[...] = jnp.broadcast_to(z * dinv, (z.shape[0], L))


_dense = pl.pallas_call(
    _dense_body,
    grid=(N // BLK,),
    in_specs=[
        pl.BlockSpec((BLK, 1), lambda i: (i, 0)),
        pl.BlockSpec((BLK, F), lambda i: (i, 0)),
        pl.BlockSpec((NC, BLK, FH), lambda i: (0, i, 0)),
        pl.BlockSpec((F, H), lambda i: (0, 0)),
        pl.BlockSpec((1, H), lambda i: (0, 0)),
        pl.BlockSpec((H, 1), lambda i: (0, 0)),
    ],
    out_specs=[
        pl.BlockSpec((BLK, 1), lambda i: (i, 0)),
        pl.BlockSpec((BLK, L), lambda i: (i, 0)),
    ],
    out_shape=[
        jax.ShapeDtypeStruct((N, 1), _f32),
        jax.ShapeDtypeStruct((N, L), _f32),
    ],
)


def _final_body(q_ref, dinv_ref, z_ref, b2_ref, batch_ref, pooled_ref):
    dinv = dinv_ref[...]
    q = q_ref[0, :, 0:1] + q_ref[1, :, 0:1]
    out = dinv * q + (2.0 * dinv * dinv) * z_ref[...] + b2_ref[0, 0]
    gids = lax.broadcasted_iota(jnp.int32, (1, G), 1)
    onehot = (batch_ref[...] == gids).astype(_f32)
    cat = jnp.concatenate([out, jnp.ones_like(out)], axis=1)
    r = lax.dot_general(onehot, cat, (((0,), (0,)), ((), ())),
                        preferred_element_type=_f32)
    pooled_ref[...] = r[:, 0:1] / jnp.maximum(r[:, 1:2], 1.0)


_final = pl.pallas_call(
    _final_body,
    out_shape=jax.ShapeDtypeStruct((G, 1), _f32),
)


def kernel(x, edge_index, edge_attr, batch, W1, b1, W2, b2):
    del edge_attr
    src2 = edge_index[0].reshape(NCHUNKS, CHUNK)
    dst2 = edge_index[1].reshape(NCHUNKS, CHUNK)
    srcb = jnp.stack([src2, src2 + N])
    zeros_nf = jnp.zeros((NP, FH), _f32)
    zeros_nl = jnp.zeros((N, L), _f32)
    ones_cl = jnp.ones((CHUNK, L), _f32)

    deg = _deg_kernel(dst2, zeros_nl, ones_cl)
    dinv, y2 = _prep(deg, x)
    acc = _agg_kernel(y2.reshape(NC * N, FH), srcb, dst2, zeros_nf)
    z, zp = _dense(dinv, x, acc, W1, b1.reshape(1, H), W2)
    q = _l2_kernel(zp, src2, dst2, zeros_nl)
    pooled = _final(q, dinv, z, b2.reshape(1, 1), batch.reshape(N, 1))
    return pooled
